# baseline (speedup 1.0000x reference)
import jax
import jax.numpy as jnp
import numpy as np
from jax import lax
from jax.experimental import pallas as pl
from jax.experimental.pallas import tpu as pltpu
from jax.experimental.pallas import tpu_sc as plsc

B = 4
N = 8192
C = 384
NP = 1024
K = 32
R, CL = 64, 128
NSC, NSUB = 2, 16
NW = NSC * NSUB
CPW = (B * NP) // NW
NCHUNK = N // 16
BIG = np.float32(3.0e38)


def _fps_body(xyz_ref, pts_ref, idx_ref, dref):
    flat = (lax.broadcasted_iota(jnp.int32, (R, CL), 0) * CL
            + lax.broadcasted_iota(jnp.int32, (R, CL), 1))
    flat_np = (lax.broadcasted_iota(jnp.int32, (8, 128), 0) * 128
               + lax.broadcasted_iota(jnp.int32, (8, 128), 1))

    init = []
    for bb in range(B):
        dref[bb] = jnp.full((R, CL), 1e10, dtype=jnp.float32)
        crow = pts_ref[bb, 0:1]
        init += [jnp.zeros((8, 128), dtype=jnp.int32),
                 jnp.int32(0),
                 crow[0:1, 0:1],
                 crow[0:1, 1:2],
                 crow[0:1, 2:3]]

    def body(i, state):
        out = []
        for bb in range(B):
            idxs, far, cx, cy, cz = state[5 * bb:5 * bb + 5]
            idxs = jnp.where(flat_np == i, far, idxs)
            Xb = xyz_ref[bb, 0]
            Yb = xyz_ref[bb, 1]
            Zb = xyz_ref[bb, 2]
            dx = Xb - cx
            dy = Yb - cy
            dz = Zb - cz
            d = dx * dx + dy * dy + dz * dz
            dv = jnp.minimum(dref[bb], d)
            dref[bb] = dv
            m = jnp.max(dv)
            far2 = jnp.min(jnp.where(dv == m, flat, N))
            crow = pts_ref[bb, pl.ds(far2, 1)]
            out += [idxs, far2,
                    crow[0:1, 0:1], crow[0:1, 1:2], crow[0:1, 2:3]]
        return tuple(out)

    state = lax.fori_loop(0, NP, body, tuple(init))
    for bb in range(B):
        idx_ref[bb] = state[5 * bb]


def _fps(xyzT, xyz):
    out = pl.pallas_call(
        _fps_body,
        out_shape=jax.ShapeDtypeStruct((B, 8, 128), jnp.int32),
        scratch_shapes=[pltpu.VMEM((B, R, CL), jnp.float32)],
    )(xyzT, xyz)
    return out.reshape(B, NP)


def _rtn_bf16(v):
    u = plsc.bitcast(v, jnp.uint32)
    lsb = (u >> 16) & jnp.uint32(1)
    u3 = (u + jnp.uint32(0x7FFF) + lsb) & jnp.uint32(0xFFFF0000)
    return plsc.bitcast(u3, jnp.float32)


def _knn_body(xyzc, fpsidx, lcxyz, kidx, kxyz, lcg,
              px, py, pz, pxb, pyb, pzb, pn, cxv, cyv, czv,
              cxvb, cyvb, czvb, cidx,
              lcbuf, kidxbuf, kxyzbuf, lcgbuf):
    w = lax.axis_index("s") * NSC + lax.axis_index("c")
    b = w // (NW // B)
    chunk = w % (NW // B)
    pltpu.sync_copy(xyzc.at[pl.ds(b * 3 * N, N)], px)
    pltpu.sync_copy(xyzc.at[pl.ds((b * 3 + 1) * N, N)], py)
    pltpu.sync_copy(xyzc.at[pl.ds((b * 3 + 2) * N, N)], pz)
    pltpu.sync_copy(fpsidx.at[pl.ds(w * CPW, CPW)], cidx)

    lane = lax.iota(jnp.int32, 16)
    boff = b * N

    def prep(cc, carry):
        base = cc * 16
        xj = px[pl.ds(base, 16)]
        yj = py[pl.ds(base, 16)]
        zj = pz[pl.ds(base, 16)]
        pxb[pl.ds(base, 16)] = _rtn_bf16(xj)
        pyb[pl.ds(base, 16)] = _rtn_bf16(yj)
        pzb[pl.ds(base, 16)] = _rtn_bf16(zj)
        pn[pl.ds(base, 16)] = (xj * xj + yj * yj) + zj * zj
        return carry

    lax.fori_loop(0, NCHUNK, prep, 0)

    for t in range(CPW // 16):
        iv = cidx[pl.ds(t * 16, 16)]
        gx = plsc.load_gather(px, [iv])
        gy = plsc.load_gather(py, [iv])
        gz = plsc.load_gather(pz, [iv])
        cxv[pl.ds(t * 16, 16)] = gx
        cyv[pl.ds(t * 16, 16)] = gy
        czv[pl.ds(t * 16, 16)] = gz
        cxvb[pl.ds(t * 16, 16)] = _rtn_bf16(gx)
        cyvb[pl.ds(t * 16, 16)] = _rtn_bf16(gy)
        czvb[pl.ds(t * 16, 16)] = _rtn_bf16(gz)
        pos = (t * 16 + lane) * 3
        plsc.store_scatter(lcbuf, [pos], gx)
        plsc.store_scatter(lcbuf, [pos + 1], gy)
        plsc.store_scatter(lcbuf, [pos + 2], gz)
        lcgbuf[pl.ds(t * 16, 16)] = iv + boff

    def per_centroid(j, carry):
        g = (j // 16) * 16
        l = j % 16
        lm = lane == l
        cx = jnp.max(jnp.where(lm, cxv[pl.ds(g, 16)], -BIG))
        cy = jnp.max(jnp.where(lm, cyv[pl.ds(g, 16)], -BIG))
        cz = jnp.max(jnp.where(lm, czv[pl.ds(g, 16)], -BIG))
        cxb = jnp.max(jnp.where(lm, cxvb[pl.ds(g, 16)], -BIG))
        cyb = jnp.max(jnp.where(lm, cyvb[pl.ds(g, 16)], -BIG))
        czb = jnp.max(jnp.where(lm, czvb[pl.ds(g, 16)], -BIG))
        nc = (cx * cx + cy * cy) + cz * cz

        def absorb(st2, d, idxvec):
            def wcond(s3):
                return jnp.any(s3[0])

            def wbody(s3):
                m3, b0, b1, i0, i1, t3 = s3
                f = plsc.all_reduce_ffs(m3)
                first = lane == f
                dval = jnp.max(jnp.where(first, d, -BIG))
                ci = jnp.max(jnp.where(first, idxvec, -1))
                eq0 = b0 == t3
                do0 = jnp.any(eq0)
                p0 = plsc.all_reduce_ffs(eq0)
                p1 = plsc.all_reduce_ffs(b1 == t3)
                m0 = (lane == p0) & do0
                m1 = (lane == p1) & jnp.logical_not(do0)
                b0 = jnp.where(m0, dval, b0)
                i0 = jnp.where(m0, ci, i0)
                b1 = jnp.where(m1, dval, b1)
                i1 = jnp.where(m1, ci, i1)
                t4 = jnp.max(jnp.maximum(b0, b1))
                m3 = m3 & jnp.logical_not(first) & (d < t4)
                return (m3, b0, b1, i0, i1, t4)

            m = d < st2[4]
            out = lax.while_loop(wcond, wbody, (m,) + st2)
            return out[1:]

        GRP = 16

        def group_body(gg, st):
            gbase = gg * (16 * GRP)
            ds_ = []
            hit = None
            thr = st[4]
            for u in range(GRP):
                off = gbase + u * 16
                xb = pxb[pl.ds(off, 16)]
                yb = pyb[pl.ds(off, 16)]
                zb = pzb[pl.ds(off, 16)]
                nj = pn[pl.ds(off, 16)]
                dot = (xb * cxb + yb * cyb) + zb * czb
                d = (jnp.float32(-2.0) * dot + nc) + nj
                ds_.append(d)
                m_u = d < thr
                hit = m_u if hit is None else (hit | m_u)

            def slow(st2):
                for u in range(GRP):
                    st2 = absorb(st2, ds_[u], gbase + u * 16 + lane)
                return st2

            return lax.cond(jnp.any(hit), slow, lambda s: s, st)

        init = (jnp.full((16,), BIG, jnp.float32),
                jnp.full((16,), BIG, jnp.float32),
                jnp.zeros((16,), jnp.int32),
                jnp.zeros((16,), jnp.int32),
                BIG)
        bd0, bd1, bi0, bi1, _ = lax.fori_loop(0, NCHUNK // GRP, group_body, init)

        def rank_body(q, rr):
            r0, r1 = rr
            qm = lane == q
            for bdh, bih in ((bd0, bi0), (bd1, bi1)):
                dq = jnp.max(jnp.where(qm, bdh, -BIG))
                iq = jnp.max(jnp.where(qm, bih, -1))
                lt0 = (dq < bd0) | ((dq == bd0) & (iq < bi0))
                lt1 = (dq < bd1) | ((dq == bd1) & (iq < bi1))
                r0 = r0 + lt0.astype(jnp.int32)
                r1 = r1 + lt1.astype(jnp.int32)
            return (r0, r1)

        r0, r1 = lax.fori_loop(0, 16, rank_body,
                               (jnp.zeros((16,), jnp.int32),
                                jnp.zeros((16,), jnp.int32)))

        gb = j * K
        pos0 = gb + r0
        pos1 = gb + r1
        plsc.store_scatter(kidxbuf, [pos0], bi0 + boff)
        plsc.store_scatter(kidxbuf, [pos1], bi1 + boff)
        for bih, posh in ((bi0, pos0), (bi1, pos1)):
            gx = plsc.load_gather(px, [bih])
            gy = plsc.load_gather(py, [bih])
            gz = plsc.load_gather(pz, [bih])
            p3 = posh * 3
            plsc.store_scatter(kxyzbuf, [p3], gx)
            plsc.store_scatter(kxyzbuf, [p3 + 1], gy)
            plsc.store_scatter(kxyzbuf, [p3 + 2], gz)
        return carry

    lax.fori_loop(0, CPW, per_centroid, 0)

    pltpu.sync_copy(lcbuf, lcxyz.at[pl.ds(w * CPW * 3, CPW * 3)])
    pltpu.sync_copy(kidxbuf, kidx.at[pl.ds(w * CPW * K, CPW * K)])
    pltpu.sync_copy(kxyzbuf, kxyz.at[pl.ds(w * CPW * K * 3, CPW * K * 3)])
    pltpu.sync_copy(lcgbuf, lcg.at[pl.ds(w * CPW, CPW)])


def _knn(xyzc, fpsidx, interpret=False):
    mesh = plsc.VectorSubcoreMesh(core_axis_name="c", subcore_axis_name="s", num_cores=NSC, num_subcores=NSUB)
    f = pl.kernel(
        _knn_body,
        out_type=(jax.ShapeDtypeStruct((B * NP * 3,), jnp.float32),
                  jax.ShapeDtypeStruct((B * NP * K,), jnp.int32),
                  jax.ShapeDtypeStruct((B * NP * K * 3,), jnp.float32),
                  jax.ShapeDtypeStruct((B * NP,), jnp.int32)),
        mesh=mesh,
        compiler_params=pltpu.CompilerParams(needs_layout_passes=False),
        scratch_types=[
            pltpu.VMEM((N,), jnp.float32),
            pltpu.VMEM((N,), jnp.float32),
            pltpu.VMEM((N,), jnp.float32),
            pltpu.VMEM((N,), jnp.float32),
            pltpu.VMEM((N,), jnp.float32),
            pltpu.VMEM((N,), jnp.float32),
            pltpu.VMEM((N,), jnp.float32),
            pltpu.VMEM((CPW + 16,), jnp.float32),
            pltpu.VMEM((CPW + 16,), jnp.float32),
            pltpu.VMEM((CPW + 16,), jnp.float32),
            pltpu.VMEM((CPW + 16,), jnp.float32),
            pltpu.VMEM((CPW + 16,), jnp.float32),
            pltpu.VMEM((CPW + 16,), jnp.float32),
            pltpu.VMEM((CPW,), jnp.int32),
            pltpu.VMEM((CPW * 3,), jnp.float32),
            pltpu.VMEM((CPW * K,), jnp.int32),
            pltpu.VMEM((CPW * K * 3,), jnp.float32),
            pltpu.VMEM((CPW,), jnp.int32),
        ],
        interpret=interpret,
    )
    return f(xyzc, fpsidx)


GCH = 128


def _gather_body(xflat, kidxf, lcgf, knnx, lcx, idxv, rows, sem):
    w = lax.axis_index("s") * NSC + lax.axis_index("c")
    rows_pw = (B * NP * K) // NW

    def loop_g(g, carry):
        base = w * rows_pw + g * GCH
        pltpu.sync_copy(kidxf.at[pl.ds(base, GCH)], idxv)
        pltpu.async_copy(xflat.at[idxv], rows, sem).wait()
        pltpu.sync_copy(rows, knnx.at[pl.ds(base, GCH)])
        return carry

    lax.fori_loop(0, rows_pw // GCH, loop_g, 0)

    base2 = w * (B * NP // NW)
    pltpu.sync_copy(lcgf.at[pl.ds(base2, GCH)], idxv)
    pltpu.async_copy(xflat.at[idxv], rows, sem).wait()
    pltpu.sync_copy(rows, lcx.at[pl.ds(base2, GCH)])


def _gather(xflat, kidxf, lcgf, interpret=False):
    mesh = plsc.VectorSubcoreMesh(core_axis_name="c", subcore_axis_name="s", num_cores=NSC, num_subcores=NSUB)
    f = pl.kernel(
        _gather_body,
        out_type=(jax.ShapeDtypeStruct((B * NP * K, C), jnp.float32),
                  jax.ShapeDtypeStruct((B * NP, C), jnp.float32)),
        mesh=mesh,
        scratch_types=[
            pltpu.VMEM((GCH,), jnp.int32),
            pltpu.VMEM((GCH, C), jnp.float32),
            pltpu.SemaphoreType.DMA,
        ],
        interpret=interpret,
    )
    return f(xflat, kidxf, lcgf)


def kernel(xyz, x):
    xyzT = jnp.transpose(xyz, (0, 2, 1))
    fps = _fps(xyzT.reshape(B, 3, R, CL), xyz)
    lcxyz, kidx, kxyz, lcg = _knn(xyzT.reshape(B * 3 * N), fps.reshape(B * NP))
    knnx, lcx = _gather(x.reshape(B * N, C), kidx, lcg)
    return (lcxyz.reshape(B, NP, 3),
            lcx.reshape(B, NP, C),
            kxyz.reshape(B, NP, K, 3),
            knnx.reshape(B, NP, K, C))


# speedup vs baseline: 1.0263x; 1.0263x over previous
import jax
import jax.numpy as jnp
import numpy as np
from jax import lax
from jax.experimental import pallas as pl
from jax.experimental.pallas import tpu as pltpu
from jax.experimental.pallas import tpu_sc as plsc

B = 4
N = 8192
C = 384
NP = 1024
K = 32
R, CL = 64, 128
NSC, NSUB = 2, 16
NW = NSC * NSUB
CPW = (B * NP) // NW
NCHUNK = N // 16
BIG = np.float32(3.0e38)


def _fps_body(xyz_ref, pts_ref, idx_ref, dref):
    flat = (lax.broadcasted_iota(jnp.int32, (R, CL), 0) * CL
            + lax.broadcasted_iota(jnp.int32, (R, CL), 1))
    flat_np = (lax.broadcasted_iota(jnp.int32, (8, 128), 0) * 128
               + lax.broadcasted_iota(jnp.int32, (8, 128), 1))

    init = []
    for bb in range(B):
        dref[bb] = jnp.full((R, CL), 1e10, dtype=jnp.float32)
        crow = pts_ref[bb, 0:1]
        init += [jnp.zeros((8, 128), dtype=jnp.int32),
                 jnp.int32(0),
                 crow[0:1, 0:1],
                 crow[0:1, 1:2],
                 crow[0:1, 2:3]]

    def body(i, state):
        out = []
        for bb in range(B):
            idxs, far, cx, cy, cz = state[5 * bb:5 * bb + 5]
            idxs = jnp.where(flat_np == i, far, idxs)
            Xb = xyz_ref[bb, 0]
            Yb = xyz_ref[bb, 1]
            Zb = xyz_ref[bb, 2]
            dx = Xb - cx
            dy = Yb - cy
            dz = Zb - cz
            d = dx * dx + dy * dy + dz * dz
            dv = jnp.minimum(dref[bb], d)
            dref[bb] = dv
            m = jnp.max(dv)
            far2 = jnp.min(jnp.where(dv == m, flat, N))
            crow = pts_ref[bb, pl.ds(far2, 1)]
            out += [idxs, far2,
                    crow[0:1, 0:1], crow[0:1, 1:2], crow[0:1, 2:3]]
        return tuple(out)

    state = lax.fori_loop(0, NP, body, tuple(init))
    for bb in range(B):
        idx_ref[bb] = state[5 * bb]


def _fps(xyzT, xyz):
    out = pl.pallas_call(
        _fps_body,
        out_shape=jax.ShapeDtypeStruct((B, 8, 128), jnp.int32),
        scratch_shapes=[pltpu.VMEM((B, R, CL), jnp.float32)],
    )(xyzT, xyz)
    return out.reshape(B, NP)


def _rtn_bf16(v):
    u = plsc.bitcast(v, jnp.uint32)
    lsb = (u >> 16) & jnp.uint32(1)
    u3 = (u + jnp.uint32(0x7FFF) + lsb) & jnp.uint32(0xFFFF0000)
    return plsc.bitcast(u3, jnp.float32)


def _knn_body(xyzc, fpsidx, lcxyz, kidx, kxyz, lcg,
              px, py, pz, pxb, pyb, pzb, pn, cxv, cyv, czv,
              cxvb, cyvb, czvb, cidx,
              lcbuf, kidxbuf, kxyzbuf, lcgbuf):
    w = lax.axis_index("s") * NSC + lax.axis_index("c")
    b = w // (NW // B)
    chunk = w % (NW // B)
    pltpu.sync_copy(xyzc.at[pl.ds(b * 3 * N, N)], px)
    pltpu.sync_copy(xyzc.at[pl.ds((b * 3 + 1) * N, N)], py)
    pltpu.sync_copy(xyzc.at[pl.ds((b * 3 + 2) * N, N)], pz)
    pltpu.sync_copy(fpsidx.at[pl.ds(w * CPW, CPW)], cidx)

    lane = lax.iota(jnp.int32, 16)
    boff = b * N

    def prep(cc, carry):
        base = cc * 16
        xj = px[pl.ds(base, 16)]
        yj = py[pl.ds(base, 16)]
        zj = pz[pl.ds(base, 16)]
        pxb[pl.ds(base, 16)] = _rtn_bf16(xj)
        pyb[pl.ds(base, 16)] = _rtn_bf16(yj)
        pzb[pl.ds(base, 16)] = _rtn_bf16(zj)
        pn[pl.ds(base, 16)] = (xj * xj + yj * yj) + zj * zj
        return carry

    lax.fori_loop(0, NCHUNK, prep, 0)

    for t in range(CPW // 16):
        iv = cidx[pl.ds(t * 16, 16)]
        gx = plsc.load_gather(px, [iv])
        gy = plsc.load_gather(py, [iv])
        gz = plsc.load_gather(pz, [iv])
        cxv[pl.ds(t * 16, 16)] = gx
        cyv[pl.ds(t * 16, 16)] = gy
        czv[pl.ds(t * 16, 16)] = gz
        cxvb[pl.ds(t * 16, 16)] = _rtn_bf16(gx)
        cyvb[pl.ds(t * 16, 16)] = _rtn_bf16(gy)
        czvb[pl.ds(t * 16, 16)] = _rtn_bf16(gz)
        pos = (t * 16 + lane) * 3
        plsc.store_scatter(lcbuf, [pos], gx)
        plsc.store_scatter(lcbuf, [pos + 1], gy)
        plsc.store_scatter(lcbuf, [pos + 2], gz)
        lcgbuf[pl.ds(t * 16, 16)] = iv + boff

    def per_centroid(j, carry):
        g = (j // 16) * 16
        l = j % 16
        lm = lane == l
        cx = jnp.max(jnp.where(lm, cxv[pl.ds(g, 16)], -BIG))
        cy = jnp.max(jnp.where(lm, cyv[pl.ds(g, 16)], -BIG))
        cz = jnp.max(jnp.where(lm, czv[pl.ds(g, 16)], -BIG))
        cxb = jnp.max(jnp.where(lm, cxvb[pl.ds(g, 16)], -BIG))
        cyb = jnp.max(jnp.where(lm, cyvb[pl.ds(g, 16)], -BIG))
        czb = jnp.max(jnp.where(lm, czvb[pl.ds(g, 16)], -BIG))
        nc = (cx * cx + cy * cy) + cz * cz

        def absorb(st2, d, idxvec):
            def wcond(s3):
                return jnp.any(s3[0])

            def wbody(s3):
                m3, b0, b1, i0, i1, t3 = s3
                f = plsc.all_reduce_ffs(m3)
                first = lane == f
                dval = jnp.max(jnp.where(first, d, -BIG))
                ci = jnp.max(jnp.where(first, idxvec, -1))
                eq0 = b0 == t3
                do0 = jnp.any(eq0)
                p0 = plsc.all_reduce_ffs(eq0)
                p1 = plsc.all_reduce_ffs(b1 == t3)
                m0 = (lane == p0) & do0
                m1 = (lane == p1) & jnp.logical_not(do0)
                b0 = jnp.where(m0, dval, b0)
                i0 = jnp.where(m0, ci, i0)
                b1 = jnp.where(m1, dval, b1)
                i1 = jnp.where(m1, ci, i1)
                t4 = jnp.max(jnp.maximum(b0, b1))
                m3 = m3 & jnp.logical_not(first) & (d < t4)
                return (m3, b0, b1, i0, i1, t4)

            m = d < st2[4]
            out = lax.while_loop(wcond, wbody, (m,) + st2)
            return out[1:]

        GRP = 8

        def group_body(gg, st):
            gbase = gg * (16 * GRP)
            ds_ = []
            hit = None
            thr = st[4]
            for u in range(GRP):
                off = gbase + u * 16
                xb = pxb[pl.ds(off, 16)]
                yb = pyb[pl.ds(off, 16)]
                zb = pzb[pl.ds(off, 16)]
                nj = pn[pl.ds(off, 16)]
                dot = (xb * cxb + yb * cyb) + zb * czb
                d = (jnp.float32(-2.0) * dot + nc) + nj
                ds_.append(d)
                m_u = d < thr
                hit = m_u if hit is None else (hit | m_u)

            def slow(st2):
                for u in range(GRP):
                    st2 = absorb(st2, ds_[u], gbase + u * 16 + lane)
                return st2

            return lax.cond(jnp.any(hit), slow, lambda s: s, st)

        init = (jnp.full((16,), BIG, jnp.float32),
                jnp.full((16,), BIG, jnp.float32),
                jnp.zeros((16,), jnp.int32),
                jnp.zeros((16,), jnp.int32),
                BIG)
        bd0, bd1, bi0, bi1, _ = lax.fori_loop(0, NCHUNK // GRP, group_body, init)

        def rank_body(q, rr):
            r0, r1 = rr
            qm = lane == q
            for bdh, bih in ((bd0, bi0), (bd1, bi1)):
                dq = jnp.max(jnp.where(qm, bdh, -BIG))
                iq = jnp.max(jnp.where(qm, bih, -1))
                lt0 = (dq < bd0) | ((dq == bd0) & (iq < bi0))
                lt1 = (dq < bd1) | ((dq == bd1) & (iq < bi1))
                r0 = r0 + lt0.astype(jnp.int32)
                r1 = r1 + lt1.astype(jnp.int32)
            return (r0, r1)

        r0, r1 = lax.fori_loop(0, 16, rank_body,
                               (jnp.zeros((16,), jnp.int32),
                                jnp.zeros((16,), jnp.int32)))

        gb = j * K
        pos0 = gb + r0
        pos1 = gb + r1
        plsc.store_scatter(kidxbuf, [pos0], bi0 + boff)
        plsc.store_scatter(kidxbuf, [pos1], bi1 + boff)
        for bih, posh in ((bi0, pos0), (bi1, pos1)):
            gx = plsc.load_gather(px, [bih])
            gy = plsc.load_gather(py, [bih])
            gz = plsc.load_gather(pz, [bih])
            p3 = posh * 3
            plsc.store_scatter(kxyzbuf, [p3], gx)
            plsc.store_scatter(kxyzbuf, [p3 + 1], gy)
            plsc.store_scatter(kxyzbuf, [p3 + 2], gz)
        return carry

    lax.fori_loop(0, CPW, per_centroid, 0)

    pltpu.sync_copy(lcbuf, lcxyz.at[pl.ds(w * CPW * 3, CPW * 3)])
    pltpu.sync_copy(kidxbuf, kidx.at[pl.ds(w * CPW * K, CPW * K)])
    pltpu.sync_copy(kxyzbuf, kxyz.at[pl.ds(w * CPW * K * 3, CPW * K * 3)])
    pltpu.sync_copy(lcgbuf, lcg.at[pl.ds(w * CPW, CPW)])


def _knn(xyzc, fpsidx, interpret=False):
    mesh = plsc.VectorSubcoreMesh(core_axis_name="c", subcore_axis_name="s", num_cores=NSC, num_subcores=NSUB)
    f = pl.kernel(
        _knn_body,
        out_type=(jax.ShapeDtypeStruct((B * NP * 3,), jnp.float32),
                  jax.ShapeDtypeStruct((B * NP * K,), jnp.int32),
                  jax.ShapeDtypeStruct((B * NP * K * 3,), jnp.float32),
                  jax.ShapeDtypeStruct((B * NP,), jnp.int32)),
        mesh=mesh,
        compiler_params=pltpu.CompilerParams(needs_layout_passes=False),
        scratch_types=[
            pltpu.VMEM((N,), jnp.float32),
            pltpu.VMEM((N,), jnp.float32),
            pltpu.VMEM((N,), jnp.float32),
            pltpu.VMEM((N,), jnp.float32),
            pltpu.VMEM((N,), jnp.float32),
            pltpu.VMEM((N,), jnp.float32),
            pltpu.VMEM((N,), jnp.float32),
            pltpu.VMEM((CPW + 16,), jnp.float32),
            pltpu.VMEM((CPW + 16,), jnp.float32),
            pltpu.VMEM((CPW + 16,), jnp.float32),
            pltpu.VMEM((CPW + 16,), jnp.float32),
            pltpu.VMEM((CPW + 16,), jnp.float32),
            pltpu.VMEM((CPW + 16,), jnp.float32),
            pltpu.VMEM((CPW,), jnp.int32),
            pltpu.VMEM((CPW * 3,), jnp.float32),
            pltpu.VMEM((CPW * K,), jnp.int32),
            pltpu.VMEM((CPW * K * 3,), jnp.float32),
            pltpu.VMEM((CPW,), jnp.int32),
        ],
        interpret=interpret,
    )
    return f(xyzc, fpsidx)


GCH = 128


def _gather_body(xflat, kidxf, lcgf, knnx, lcx, idxv, rows, sem):
    w = lax.axis_index("s") * NSC + lax.axis_index("c")
    rows_pw = (B * NP * K) // NW

    def loop_g(g, carry):
        base = w * rows_pw + g * GCH
        pltpu.sync_copy(kidxf.at[pl.ds(base, GCH)], idxv)
        pltpu.async_copy(xflat.at[idxv], rows, sem).wait()
        pltpu.sync_copy(rows, knnx.at[pl.ds(base, GCH)])
        return carry

    lax.fori_loop(0, rows_pw // GCH, loop_g, 0)

    base2 = w * (B * NP // NW)
    pltpu.sync_copy(lcgf.at[pl.ds(base2, GCH)], idxv)
    pltpu.async_copy(xflat.at[idxv], rows, sem).wait()
    pltpu.sync_copy(rows, lcx.at[pl.ds(base2, GCH)])


def _gather(xflat, kidxf, lcgf, interpret=False):
    mesh = plsc.VectorSubcoreMesh(core_axis_name="c", subcore_axis_name="s", num_cores=NSC, num_subcores=NSUB)
    f = pl.kernel(
        _gather_body,
        out_type=(jax.ShapeDtypeStruct((B * NP * K, C), jnp.float32),
                  jax.ShapeDtypeStruct((B * NP, C), jnp.float32)),
        mesh=mesh,
        scratch_types=[
            pltpu.VMEM((GCH,), jnp.int32),
            pltpu.VMEM((GCH, C), jnp.float32),
            pltpu.SemaphoreType.DMA,
        ],
        interpret=interpret,
    )
    return f(xflat, kidxf, lcgf)


def kernel(xyz, x):
    xyzT = jnp.transpose(xyz, (0, 2, 1))
    fps = _fps(xyzT.reshape(B, 3, R, CL), xyz)
    lcxyz, kidx, kxyz, lcg = _knn(xyzT.reshape(B * 3 * N), fps.reshape(B * NP))
    knnx, lcx = _gather(x.reshape(B * N, C), kidx, lcg)
    return (lcxyz.reshape(B, NP, 3),
            lcx.reshape(B, NP, C),
            kxyz.reshape(B, NP, K, 3),
            knnx.reshape(B, NP, K, C))
